# baseline (speedup 1.0000x reference)
import functools

import jax
import jax.numpy as jnp
from jax.experimental import pallas as pl
from jax.experimental.pallas import tpu as pltpu
from jax.experimental.pallas import tpu_sc as plsc

N = 2048
D = 768
H = 12
DH = 64
E = 64
FF = 512
BT = 64
NB = 96
PAD = NB * BT
BS = 256
BQ = 256

NW = 16
TPW = N // NW
GPW = TPW // 16
SEG = PAD // NW
HALF = PAD // 2
CH = 64

_INTERPRET = False


def _preattn_body(x_ref, w_ref, wq_ref, wk_ref, wv_ref, cos_ref, sin_ref,
                  q_ref, k_ref, v_ref):
    x = x_ref[...]
    var = jnp.mean(x * x, axis=1, keepdims=True)
    h = x * jax.lax.rsqrt(var + 1e-5) * w_ref[...]
    cos = cos_ref[...]
    sin = sin_ref[...]

    def rope(t):
        parts = []
        for hh in range(H):
            th = t[:, hh * DH:(hh + 1) * DH]
            rh = jnp.concatenate([-th[:, DH // 2:], th[:, :DH // 2]], axis=1)
            ch = cos[:, hh * DH:(hh + 1) * DH]
            sh = sin[:, hh * DH:(hh + 1) * DH]
            parts.append(th * ch + rh * sh)
        return jnp.concatenate(parts, axis=1)

    q = jnp.dot(h, wq_ref[...], preferred_element_type=jnp.float32)
    k = jnp.dot(h, wk_ref[...], preferred_element_type=jnp.float32)
    v = jnp.dot(h, wv_ref[...], preferred_element_type=jnp.float32)
    q_ref[...] = rope(q)
    k_ref[...] = rope(k)
    v_ref[...] = v


def _attn_body(q_ref, k_ref, v_ref, o_ref):
    qi = pl.program_id(1)
    q = q_ref[...]
    k = k_ref[...]
    v = v_ref[...]
    rows = qi * BQ + jax.lax.broadcasted_iota(jnp.int32, (BQ, N), 0)
    cols = jax.lax.broadcasted_iota(jnp.int32, (BQ, N), 1)
    mask = cols <= rows
    outs = []
    for hh in range(2):
        qh = q[:, hh * DH:(hh + 1) * DH]
        kh = k[:, hh * DH:(hh + 1) * DH]
        vh = v[:, hh * DH:(hh + 1) * DH]
        s = jax.lax.dot_general(qh, kh, (((1,), (1,)), ((), ())),
                                preferred_element_type=jnp.float32) * 0.125
        s = jnp.where(mask, s, jnp.float32(-1e30))
        m = jnp.max(s, axis=1, keepdims=True)
        p = jnp.exp(s - m)
        l = jnp.sum(p, axis=1, keepdims=True)
        ctx = jax.lax.dot_general(p, vh, (((1,), (0,)), ((), ())),
                                  preferred_element_type=jnp.float32)
        outs.append(ctx / l)
    o_ref[...] = jnp.concatenate(outs, axis=1)


def _postattn_body(ctx_ref, x_ref, wo_ref, ln2_ref, wr_ref, wsg_ref, wsd_ref,
                   base_ref, h2_ref, eid_ref, gate_ref):
    xa = x_ref[...] + jnp.dot(ctx_ref[...], wo_ref[...],
                              preferred_element_type=jnp.float32)
    var = jnp.mean(xa * xa, axis=1, keepdims=True)
    h2 = xa * jax.lax.rsqrt(var + 1e-5) * ln2_ref[...]
    logits = jnp.dot(h2, wr_ref[...], preferred_element_type=jnp.float32)
    mx = jnp.max(logits, axis=1, keepdims=True)
    eid_ref[...] = jnp.argmax(logits, axis=1).astype(jnp.int32)[:, None]
    gate_ref[...] = 1.0 / jnp.sum(jnp.exp(logits - mx), axis=1,
                                  keepdims=True)
    gu = jnp.dot(h2, wsg_ref[...], preferred_element_type=jnp.float32)
    g = gu[:, :FF]
    u = gu[:, FF:]
    shared = jnp.dot(jax.nn.silu(g) * u, wsd_ref[...],
                     preferred_element_type=jnp.float32)
    base_ref[...] = xa + shared
    h2_ref[...] = h2


def _expert_ffn_body(be_ref, xs_ref, weg_ref, wed_ref, gate_ref, o_ref):
    del be_ref
    xb = xs_ref[...]
    gu = jnp.dot(xb, weg_ref[0], preferred_element_type=jnp.float32)
    g = gu[:, :FF]
    u = gu[:, FF:]
    y = jnp.dot(jax.nn.silu(g) * u, wed_ref[0],
                preferred_element_type=jnp.float32)
    o_ref[...] = y * gate_ref[...]


def _add_body(a_ref, b_ref, o_ref):
    o_ref[...] = a_ref[...] + b_ref[...]


def _routing_body(eid_hbm, gate_hbm, h2_hbm,
                  xs_hbm, inv_hbm, gout_hbm, be_hbm,
                  eidv, gatev, skv, stv, rkv, sgv, histg, t2v, prv, psv, bsv,
                  bev, tokfull, gatefull, invseg, tmpi, tmpf, desti, destf,
                  cidx, rows, t2s, tokS, gtS, sem):
    i32 = jnp.int32
    c = jax.lax.axis_index("c")
    s = jax.lax.axis_index("s")
    iota = jax.lax.iota(i32, 16)
    zeros16 = jnp.zeros((16,), i32)

    pltpu.sync_copy(eid_hbm.at[pl.ds(s * TPW, TPW)], eidv)
    pltpu.sync_copy(gate_hbm.at[pl.ds(s * TPW, TPW)], gatev)
    for g in range(GPW):
        for k in range(E // 16):
            histg[g, pl.ds(16 * k, 16)] = zeros16
    for g in range(GPW):
        ev = eidv[pl.ds(16 * g, 16)]
        sk, sp = plsc.sort_key_val(ev, iota)
        skv[g, pl.ds(0, 16)] = sk
        tok = s * TPW + 16 * g + sp
        sg = plsc.load_gather(gatev, [sp + 16 * g])
        prev = plsc.load_gather(skv, [jnp.full((16,), g, i32),
                                      jnp.maximum(iota - 1, 0)])
        nxt = plsc.load_gather(skv, [jnp.full((16,), g, i32),
                                     jnp.minimum(iota + 1, 15)])
        bound = (iota == 0) | (sk != prev)
        rst = plsc.cummax(jnp.where(bound, iota, 0))
        rk = iota - rst
        is_last = (iota == 15) | (sk != nxt)
        plsc.store_scatter(histg, [jnp.full((16,), g, i32), sk], rk + 1,
                           mask=is_last)
        stv[g, pl.ds(0, 16)] = tok
        rkv[g, pl.ds(0, 16)] = rk
        sgv[g, pl.ds(0, 16)] = sg
    pltpu.sync_copy(histg, t2s.at[pl.ds(s * GPW, GPW)])

    def zero_body(j, _):
        tokfull[pl.ds(16 * j, 16)] = zeros16
        gatefull[pl.ds(16 * j, 16)] = zeros16.astype(jnp.float32)
        return 0

    jax.lax.fori_loop(0, PAD // 16, zero_body, 0)
    plsc.subcore_barrier()

    pltpu.sync_copy(t2s, t2v)
    zero4 = tuple(jnp.zeros((16,), i32) for _ in range(4))

    def scan_row(r, carry):
        base, tot = carry[:4], carry[4:]
        pred = r < s * GPW
        nb = []
        nt = []
        for k in range(4):
            row = t2v[r, pl.ds(16 * k, 16)]
            nb.append(base[k] + jnp.where(pred, row, 0))
            nt.append(tot[k] + row)
        return tuple(nb) + tuple(nt)

    acc = jax.lax.fori_loop(0, NW * GPW, scan_row, zero4 + zero4)
    base, ctot = list(acc[:4]), list(acc[4:])
    for g in range(GPW):
        for k in range(4):
            prv[g, pl.ds(16 * k, 16)] = base[k]
            base[k] = base[k] + t2v[s * GPW + g, pl.ds(16 * k, 16)]

    carry = jnp.zeros((), i32)
    for k in range(4):
        padk = ((ctot[k] + (BT - 1)) >> 6) << 6
        incl = plsc.cumsum(padk)
        excl = incl - padk + carry
        psv[pl.ds(16 * k, 16)] = excl
        bsv[pl.ds(16 * k, 16)] = excl >> 6
        carry = carry + jnp.sum(padk)

    for g in range(GPW):
        ev = skv[g, pl.ds(0, 16)]
        slot = (plsc.load_gather(psv, [ev])
                + plsc.load_gather(prv, [jnp.full((16,), g, i32), ev])
                + rkv[g, pl.ds(0, 16)])
        tok = stv[g, pl.ds(0, 16)]
        plsc.store_scatter(tokfull, [slot], tok + 1)
        plsc.store_scatter(gatefull, [slot], sgv[g, pl.ds(0, 16)])
        plsc.store_scatter(invseg, [tok - s * TPW], slot)
    pltpu.sync_copy(invseg, inv_hbm.at[pl.ds(s * TPW, TPW)])
    pltpu.sync_copy(tokfull, tokS.at[s])
    pltpu.sync_copy(gatefull, gtS.at[s])

    @pl.when((c == 0) & (s == 0))
    def _():
        def be_row(e, carry):
            bs_e = plsc.load_gather(bsv, [jnp.full((16,), e, i32)])
            return tuple(carry[kb] + (bs_e <= (iota + 16 * kb)).astype(i32)
                         for kb in range(NB // 16))

        cnt = jax.lax.fori_loop(
            0, E, be_row, tuple(jnp.zeros((16,), i32)
                                for _ in range(NB // 16)))
        for kb in range(NB // 16):
            bev[pl.ds(16 * kb, 16)] = jnp.minimum(cnt[kb] - 1, E - 1)
        pltpu.sync_copy(bev, be_hbm)

    plsc.subcore_barrier()

    segsz = HALF // NW
    pltpu.sync_copy(tokS.at[:, pl.ds(SEG * s, SEG)], tmpi)
    pltpu.sync_copy(gtS.at[:, pl.ds(SEG * s, SEG)], tmpf)
    for j in range(SEG // 16):
        acc_i = zeros16
        acc_f = zeros16.astype(jnp.float32)
        for w in range(NW):
            acc_i = acc_i + tmpi[w, pl.ds(16 * j, 16)]
            acc_f = acc_f + tmpf[w, pl.ds(16 * j, 16)]
        desti[pl.ds(16 * j, 16)] = jnp.maximum(acc_i - 1, 0)
        destf[pl.ds(16 * j, 16)] = acc_f
    off = segsz * c
    seg = SEG * s + off
    pltpu.sync_copy(destf.at[pl.ds(off, segsz)],
                    gout_hbm.at[pl.ds(seg, segsz)])
    for ch in range(segsz // CH):
        for k in range(CH // 16):
            cidx[pl.ds(16 * k, 16)] = desti[pl.ds(off + CH * ch + 16 * k, 16)]
        pltpu.async_copy(h2_hbm.at[cidx], rows, sem).wait()
        pltpu.sync_copy(rows, xs_hbm.at[pl.ds(seg + CH * ch, CH)])


def _scatter_body(ys_hbm, inv_hbm, yt_hbm, invv, cidx, rows, sem):
    c = jax.lax.axis_index("c")
    s = jax.lax.axis_index("s")
    base = (16 * c + s) * (N // 32)
    pltpu.sync_copy(inv_hbm.at[pl.ds(base, N // 32)], invv)
    for ch in range((N // 32) // CH):
        for k in range(CH // 16):
            cidx[pl.ds(16 * k, 16)] = invv[pl.ds(CH * ch + 16 * k, 16)]
        pltpu.async_copy(ys_hbm.at[cidx], rows, sem).wait()
        pltpu.sync_copy(rows, yt_hbm.at[pl.ds(base + CH * ch, CH)])


def kernel(x, position_ids, ln1_w, ln2_w, Wq, Wk, Wv, Wo, Wr, Wsg, Wsd,
           Weg, Wed):
    f32 = jnp.float32
    i32 = jnp.int32
    xf = x.reshape(N, D)

    inv_freq = 1.0 / (10000.0 ** (jnp.arange(0, DH, 2, dtype=f32) / DH))
    freqs = position_ids.reshape(N, 1).astype(f32) * inv_freq[None, :]
    emb = jnp.concatenate([freqs, freqs], axis=1)
    cos_t = jnp.tile(jnp.cos(emb), (1, H))
    sin_t = jnp.tile(jnp.sin(emb), (1, H))

    full = lambda shape: pl.BlockSpec(shape, lambda i: (0,) * len(shape))
    rowblk = lambda w: pl.BlockSpec((BS, w), lambda i: (i, 0))

    q, k, v = pl.pallas_call(
        _preattn_body,
        grid=(N // BS,),
        in_specs=[rowblk(D), full((1, D)), full((D, D)), full((D, D)),
                  full((D, D)), rowblk(D), rowblk(D)],
        out_specs=[rowblk(D)] * 3,
        out_shape=[jax.ShapeDtypeStruct((N, D), f32)] * 3,
        interpret=_INTERPRET,
    )(xf, ln1_w.reshape(1, D), Wq, Wk, Wv, cos_t, sin_t)

    ctx = pl.pallas_call(
        _attn_body,
        grid=(H // 2, N // BQ),
        in_specs=[
            pl.BlockSpec((BQ, 2 * DH), lambda p, i: (i, p)),
            pl.BlockSpec((N, 2 * DH), lambda p, i: (0, p)),
            pl.BlockSpec((N, 2 * DH), lambda p, i: (0, p)),
        ],
        out_specs=pl.BlockSpec((BQ, 2 * DH), lambda p, i: (i, p)),
        out_shape=jax.ShapeDtypeStruct((N, D), f32),
        interpret=_INTERPRET,
    )(q, k, v)

    base, h2, eid2, gate2 = pl.pallas_call(
        _postattn_body,
        grid=(N // BS,),
        in_specs=[rowblk(D), rowblk(D), full((D, D)), full((1, D)),
                  full((D, E)), full((D, 2 * FF)), full((FF, D))],
        out_specs=[rowblk(D), rowblk(D), rowblk(1), rowblk(1)],
        out_shape=[jax.ShapeDtypeStruct((N, D), f32),
                   jax.ShapeDtypeStruct((N, D), f32),
                   jax.ShapeDtypeStruct((N, 1), i32),
                   jax.ShapeDtypeStruct((N, 1), f32)],
        interpret=_INTERPRET,
    )(ctx, xf, Wo, ln2_w.reshape(1, D), Wr, Wsg, Wsd)

    mesh = plsc.VectorSubcoreMesh(core_axis_name="c", subcore_axis_name="s")
    sc_route = functools.partial(
        pl.kernel,
        out_type=[jax.ShapeDtypeStruct((PAD, D), f32),
                  jax.ShapeDtypeStruct((N,), i32),
                  jax.ShapeDtypeStruct((PAD,), f32),
                  jax.ShapeDtypeStruct((NB,), i32)],
        mesh=mesh,
        compiler_params=pltpu.CompilerParams(needs_layout_passes=False),
        scratch_types=[
            pltpu.VMEM((TPW,), i32),
            pltpu.VMEM((TPW,), f32),
            pltpu.VMEM((GPW, 16), i32),
            pltpu.VMEM((GPW, 16), i32),
            pltpu.VMEM((GPW, 16), i32),
            pltpu.VMEM((GPW, 16), f32),
            pltpu.VMEM((GPW, E), i32),
            pltpu.VMEM((NW * GPW, E), i32),
            pltpu.VMEM((GPW, E), i32),
            pltpu.VMEM((E,), i32),
            pltpu.VMEM((E,), i32),
            pltpu.VMEM((NB,), i32),
            pltpu.VMEM((PAD,), i32),
            pltpu.VMEM((PAD,), f32),
            pltpu.VMEM((TPW,), i32),
            pltpu.VMEM((NW, SEG), i32),
            pltpu.VMEM((NW, SEG), f32),
            pltpu.VMEM((SEG,), i32),
            pltpu.VMEM((SEG,), f32),
            pltpu.VMEM((CH,), i32),
            pltpu.VMEM((CH, D), f32),
            pltpu.VMEM_SHARED((NW * GPW, E), i32),
            pltpu.VMEM_SHARED((NW, PAD), i32),
            pltpu.VMEM_SHARED((NW, PAD), f32),
            pltpu.SemaphoreType.DMA,
        ],
    )(_routing_body)
    x_sorted, inv, gate_s, block_expert = sc_route(
        eid2.reshape(N), gate2.reshape(N), h2)

    ys = pl.pallas_call(
        _expert_ffn_body,
        grid_spec=pltpu.PrefetchScalarGridSpec(
            num_scalar_prefetch=1,
            grid=(NB,),
            in_specs=[
                pl.BlockSpec((BT, D), lambda b, be: (b, 0)),
                pl.BlockSpec((1, D, 2 * FF), lambda b, be: (be[b], 0, 0)),
                pl.BlockSpec((1, FF, D), lambda b, be: (be[b], 0, 0)),
                pl.BlockSpec((BT, 1), lambda b, be: (b, 0)),
            ],
            out_specs=pl.BlockSpec((BT, D), lambda b, be: (b, 0)),
        ),
        out_shape=jax.ShapeDtypeStruct((PAD, D), f32),
        interpret=_INTERPRET,
    )(block_expert, x_sorted, Weg, Wed, gate_s.reshape(PAD, 1))

    sc_scatter = functools.partial(
        pl.kernel,
        out_type=jax.ShapeDtypeStruct((N, D), f32),
        mesh=mesh,
        compiler_params=pltpu.CompilerParams(needs_layout_passes=False),
        scratch_types=[
            pltpu.VMEM((N // 32,), i32),
            pltpu.VMEM((CH,), i32),
            pltpu.VMEM((CH, D), f32),
            pltpu.SemaphoreType.DMA,
        ],
    )(_scatter_body)
    y_tok = sc_scatter(ys, inv)

    out = pl.pallas_call(
        _add_body,
        grid=(N // BS,),
        in_specs=[rowblk(D), rowblk(D)],
        out_specs=rowblk(D),
        out_shape=jax.ShapeDtypeStruct((N, D), f32),
        interpret=_INTERPRET,
    )(base, y_tok)
    return out.reshape(1, N, D)


# speedup vs baseline: 1.4337x; 1.4337x over previous
import functools

import jax
import jax.numpy as jnp
from jax.experimental import pallas as pl
from jax.experimental.pallas import tpu as pltpu
from jax.experimental.pallas import tpu_sc as plsc

N = 2048
D = 768
H = 12
DH = 64
E = 64
FF = 512
BT = 64
NB = 96
PAD = NB * BT
BS = 256
BQ = 256

NW = 16
TPW = N // NW
GPW = TPW // 16
SEG = PAD // NW
HALF = PAD // 2
CH = 64

_INTERPRET = False


def _preattn_body(x_ref, w_ref, wq_ref, wk_ref, wv_ref, cos_ref, sin_ref,
                  q_ref, k_ref, v_ref):
    x = x_ref[...]
    var = jnp.mean(x * x, axis=1, keepdims=True)
    h = x * jax.lax.rsqrt(var + 1e-5) * w_ref[...]
    cos = cos_ref[...]
    sin = sin_ref[...]

    def rope(t):
        parts = []
        for hh in range(H):
            th = t[:, hh * DH:(hh + 1) * DH]
            rh = jnp.concatenate([-th[:, DH // 2:], th[:, :DH // 2]], axis=1)
            ch = cos[:, hh * DH:(hh + 1) * DH]
            sh = sin[:, hh * DH:(hh + 1) * DH]
            parts.append(th * ch + rh * sh)
        return jnp.concatenate(parts, axis=1)

    q = jnp.dot(h, wq_ref[...], preferred_element_type=jnp.float32)
    k = jnp.dot(h, wk_ref[...], preferred_element_type=jnp.float32)
    v = jnp.dot(h, wv_ref[...], preferred_element_type=jnp.float32)
    q_ref[...] = rope(q)
    k_ref[...] = rope(k)
    v_ref[...] = v


def _attn_body(q_ref, k_ref, v_ref, o_ref):
    qi = pl.program_id(1)
    q = q_ref[...]
    k = k_ref[...]
    v = v_ref[...]
    rows = qi * BQ + jax.lax.broadcasted_iota(jnp.int32, (BQ, N), 0)
    cols = jax.lax.broadcasted_iota(jnp.int32, (BQ, N), 1)
    mask = cols <= rows
    outs = []
    for hh in range(2):
        qh = q[:, hh * DH:(hh + 1) * DH]
        kh = k[:, hh * DH:(hh + 1) * DH]
        vh = v[:, hh * DH:(hh + 1) * DH]
        s = jax.lax.dot_general(qh, kh, (((1,), (1,)), ((), ())),
                                preferred_element_type=jnp.float32) * 0.125
        s = jnp.where(mask, s, jnp.float32(-1e30))
        m = jnp.max(s, axis=1, keepdims=True)
        p = jnp.exp(s - m)
        l = jnp.sum(p, axis=1, keepdims=True)
        ctx = jax.lax.dot_general(p, vh, (((1,), (0,)), ((), ())),
                                  preferred_element_type=jnp.float32)
        outs.append(ctx / l)
    o_ref[...] = jnp.concatenate(outs, axis=1)


def _postattn_body(ctx_ref, x_ref, wo_ref, ln2_ref, wr_ref, wsg_ref, wsd_ref,
                   base_ref, h2_ref, eid_ref, gate_ref):
    xa = x_ref[...] + jnp.dot(ctx_ref[...], wo_ref[...],
                              preferred_element_type=jnp.float32)
    var = jnp.mean(xa * xa, axis=1, keepdims=True)
    h2 = xa * jax.lax.rsqrt(var + 1e-5) * ln2_ref[...]
    logits = jnp.dot(h2, wr_ref[...], preferred_element_type=jnp.float32)
    mx = jnp.max(logits, axis=1, keepdims=True)
    eid_ref[...] = jnp.argmax(logits, axis=1).astype(jnp.int32)[:, None]
    gate_ref[...] = 1.0 / jnp.sum(jnp.exp(logits - mx), axis=1,
                                  keepdims=True)
    gu = jnp.dot(h2, wsg_ref[...], preferred_element_type=jnp.float32)
    g = gu[:, :FF]
    u = gu[:, FF:]
    shared = jnp.dot(jax.nn.silu(g) * u, wsd_ref[...],
                     preferred_element_type=jnp.float32)
    base_ref[...] = xa + shared
    h2_ref[...] = h2


def _expert_ffn_body(be_ref, xs_ref, weg_ref, wed_ref, gate_ref, o_ref):
    del be_ref
    xb = xs_ref[...]
    gu = jnp.dot(xb, weg_ref[0], preferred_element_type=jnp.float32)
    g = gu[:, :FF]
    u = gu[:, FF:]
    y = jnp.dot(jax.nn.silu(g) * u, wed_ref[0],
                preferred_element_type=jnp.float32)
    o_ref[...] = y * gate_ref[...]


def _add_body(a_ref, b_ref, o_ref):
    o_ref[...] = a_ref[...] + b_ref[...]


def _routing_body(eid_hbm, gate_hbm, h2_hbm,
                  xs_hbm, inv_hbm, gout_hbm, be_hbm,
                  eidv, gatev, skv, stv, rkv, sgv, histg, t2v, prv, psv, bsv,
                  bev, tokfull, gatefull, invseg, tmpi, tmpf, desti, destf,
                  cidx, rows, t2s, tokS, gtS, sem):
    i32 = jnp.int32
    c = jax.lax.axis_index("c")
    s = jax.lax.axis_index("s")
    iota = jax.lax.iota(i32, 16)
    zeros16 = jnp.zeros((16,), i32)

    pltpu.sync_copy(eid_hbm.at[pl.ds(s * TPW, TPW)], eidv)
    pltpu.sync_copy(gate_hbm.at[pl.ds(s * TPW, TPW)], gatev)
    for g in range(GPW):
        for k in range(E // 16):
            histg[g, pl.ds(16 * k, 16)] = zeros16
    for g in range(GPW):
        ev = eidv[pl.ds(16 * g, 16)]
        sk, sp = plsc.sort_key_val(ev, iota)
        skv[g, pl.ds(0, 16)] = sk
        tok = s * TPW + 16 * g + sp
        sg = plsc.load_gather(gatev, [sp + 16 * g])
        prev = plsc.load_gather(skv, [jnp.full((16,), g, i32),
                                      jnp.maximum(iota - 1, 0)])
        nxt = plsc.load_gather(skv, [jnp.full((16,), g, i32),
                                     jnp.minimum(iota + 1, 15)])
        bound = (iota == 0) | (sk != prev)
        rst = plsc.cummax(jnp.where(bound, iota, 0))
        rk = iota - rst
        is_last = (iota == 15) | (sk != nxt)
        plsc.store_scatter(histg, [jnp.full((16,), g, i32), sk], rk + 1,
                           mask=is_last)
        stv[g, pl.ds(0, 16)] = tok
        rkv[g, pl.ds(0, 16)] = rk
        sgv[g, pl.ds(0, 16)] = sg
    pltpu.sync_copy(histg, t2s.at[pl.ds(s * GPW, GPW)])

    def zero_body(j, _):
        tokfull[pl.ds(16 * j, 16)] = zeros16
        gatefull[pl.ds(16 * j, 16)] = zeros16.astype(jnp.float32)
        return 0

    jax.lax.fori_loop(0, PAD // 16, zero_body, 0)
    plsc.subcore_barrier()

    pltpu.sync_copy(t2s, t2v)
    zero4 = tuple(jnp.zeros((16,), i32) for _ in range(4))

    def scan_row(r, carry):
        base, tot = carry[:4], carry[4:]
        pred = r < s * GPW
        nb = []
        nt = []
        for k in range(4):
            row = t2v[r, pl.ds(16 * k, 16)]
            nb.append(base[k] + jnp.where(pred, row, 0))
            nt.append(tot[k] + row)
        return tuple(nb) + tuple(nt)

    acc = jax.lax.fori_loop(0, NW * GPW, scan_row, zero4 + zero4)
    base, ctot = list(acc[:4]), list(acc[4:])
    for g in range(GPW):
        for k in range(4):
            prv[g, pl.ds(16 * k, 16)] = base[k]
            base[k] = base[k] + t2v[s * GPW + g, pl.ds(16 * k, 16)]

    carry = jnp.zeros((), i32)
    for k in range(4):
        padk = ((ctot[k] + (BT - 1)) >> 6) << 6
        incl = plsc.cumsum(padk)
        excl = incl - padk + carry
        psv[pl.ds(16 * k, 16)] = excl
        bsv[pl.ds(16 * k, 16)] = excl >> 6
        carry = carry + jnp.sum(padk)

    for g in range(GPW):
        ev = skv[g, pl.ds(0, 16)]
        slot = (plsc.load_gather(psv, [ev])
                + plsc.load_gather(prv, [jnp.full((16,), g, i32), ev])
                + rkv[g, pl.ds(0, 16)])
        tok = stv[g, pl.ds(0, 16)]
        plsc.store_scatter(tokfull, [slot], tok + 1)
        plsc.store_scatter(gatefull, [slot], sgv[g, pl.ds(0, 16)])
        plsc.store_scatter(invseg, [tok - s * TPW], slot)
    pltpu.sync_copy(invseg, inv_hbm.at[pl.ds(s * TPW, TPW)])
    pltpu.sync_copy(tokfull, tokS.at[s])
    pltpu.sync_copy(gatefull, gtS.at[s])

    @pl.when((c == 0) & (s == 0))
    def _():
        def be_row(e, carry):
            bs_e = plsc.load_gather(bsv, [jnp.full((16,), e, i32)])
            return tuple(carry[kb] + (bs_e <= (iota + 16 * kb)).astype(i32)
                         for kb in range(NB // 16))

        cnt = jax.lax.fori_loop(
            0, E, be_row, tuple(jnp.zeros((16,), i32)
                                for _ in range(NB // 16)))
        for kb in range(NB // 16):
            bev[pl.ds(16 * kb, 16)] = jnp.minimum(cnt[kb] - 1, E - 1)
        pltpu.sync_copy(bev, be_hbm)

    plsc.subcore_barrier()

    segsz = HALF // NW
    pltpu.sync_copy(tokS.at[:, pl.ds(SEG * s, SEG)], tmpi)
    pltpu.sync_copy(gtS.at[:, pl.ds(SEG * s, SEG)], tmpf)
    for j in range(SEG // 16):
        acc_i = zeros16
        acc_f = zeros16.astype(jnp.float32)
        for w in range(NW):
            acc_i = acc_i + tmpi[w, pl.ds(16 * j, 16)]
            acc_f = acc_f + tmpf[w, pl.ds(16 * j, 16)]
        slotvec = SEG * s + 16 * j + iota
        desti[pl.ds(16 * j, 16)] = jnp.where(acc_i == 0, slotvec & (N - 1),
                                             acc_i - 1)
        destf[pl.ds(16 * j, 16)] = acc_f
    off = segsz * c
    seg = SEG * s + off
    pltpu.sync_copy(destf.at[pl.ds(off, segsz)],
                    gout_hbm.at[pl.ds(seg, segsz)])
    for ch in range(segsz // CH):
        for k in range(CH // 16):
            cidx[pl.ds(16 * k, 16)] = desti[pl.ds(off + CH * ch + 16 * k, 16)]
        pltpu.async_copy(h2_hbm.at[cidx], rows, sem).wait()
        pltpu.sync_copy(rows, xs_hbm.at[pl.ds(seg + CH * ch, CH)])


def _scatter_body(ys_hbm, inv_hbm, yt_hbm, invv, cidx, rows, sem):
    c = jax.lax.axis_index("c")
    s = jax.lax.axis_index("s")
    base = (16 * c + s) * (N // 32)
    pltpu.sync_copy(inv_hbm.at[pl.ds(base, N // 32)], invv)
    for ch in range((N // 32) // CH):
        for k in range(CH // 16):
            cidx[pl.ds(16 * k, 16)] = invv[pl.ds(CH * ch + 16 * k, 16)]
        pltpu.async_copy(ys_hbm.at[cidx], rows, sem).wait()
        pltpu.sync_copy(rows, yt_hbm.at[pl.ds(base + CH * ch, CH)])


def kernel(x, position_ids, ln1_w, ln2_w, Wq, Wk, Wv, Wo, Wr, Wsg, Wsd,
           Weg, Wed):
    f32 = jnp.float32
    i32 = jnp.int32
    xf = x.reshape(N, D)

    inv_freq = 1.0 / (10000.0 ** (jnp.arange(0, DH, 2, dtype=f32) / DH))
    freqs = position_ids.reshape(N, 1).astype(f32) * inv_freq[None, :]
    emb = jnp.concatenate([freqs, freqs], axis=1)
    cos_t = jnp.tile(jnp.cos(emb), (1, H))
    sin_t = jnp.tile(jnp.sin(emb), (1, H))

    full = lambda shape: pl.BlockSpec(shape, lambda i: (0,) * len(shape))
    rowblk = lambda w: pl.BlockSpec((BS, w), lambda i: (i, 0))

    q, k, v = pl.pallas_call(
        _preattn_body,
        grid=(N // BS,),
        in_specs=[rowblk(D), full((1, D)), full((D, D)), full((D, D)),
                  full((D, D)), rowblk(D), rowblk(D)],
        out_specs=[rowblk(D)] * 3,
        out_shape=[jax.ShapeDtypeStruct((N, D), f32)] * 3,
        interpret=_INTERPRET,
    )(xf, ln1_w.reshape(1, D), Wq, Wk, Wv, cos_t, sin_t)

    ctx = pl.pallas_call(
        _attn_body,
        grid=(H // 2, N // BQ),
        in_specs=[
            pl.BlockSpec((BQ, 2 * DH), lambda p, i: (i, p)),
            pl.BlockSpec((N, 2 * DH), lambda p, i: (0, p)),
            pl.BlockSpec((N, 2 * DH), lambda p, i: (0, p)),
        ],
        out_specs=pl.BlockSpec((BQ, 2 * DH), lambda p, i: (i, p)),
        out_shape=jax.ShapeDtypeStruct((N, D), f32),
        interpret=_INTERPRET,
    )(q, k, v)

    base, h2, eid2, gate2 = pl.pallas_call(
        _postattn_body,
        grid=(N // BS,),
        in_specs=[rowblk(D), rowblk(D), full((D, D)), full((1, D)),
                  full((D, E)), full((D, 2 * FF)), full((FF, D))],
        out_specs=[rowblk(D), rowblk(D), rowblk(1), rowblk(1)],
        out_shape=[jax.ShapeDtypeStruct((N, D), f32),
                   jax.ShapeDtypeStruct((N, D), f32),
                   jax.ShapeDtypeStruct((N, 1), i32),
                   jax.ShapeDtypeStruct((N, 1), f32)],
        interpret=_INTERPRET,
    )(ctx, xf, Wo, ln2_w.reshape(1, D), Wr, Wsg, Wsd)

    mesh = plsc.VectorSubcoreMesh(core_axis_name="c", subcore_axis_name="s")
    sc_route = functools.partial(
        pl.kernel,
        out_type=[jax.ShapeDtypeStruct((PAD, D), f32),
                  jax.ShapeDtypeStruct((N,), i32),
                  jax.ShapeDtypeStruct((PAD,), f32),
                  jax.ShapeDtypeStruct((NB,), i32)],
        mesh=mesh,
        compiler_params=pltpu.CompilerParams(needs_layout_passes=False),
        scratch_types=[
            pltpu.VMEM((TPW,), i32),
            pltpu.VMEM((TPW,), f32),
            pltpu.VMEM((GPW, 16), i32),
            pltpu.VMEM((GPW, 16), i32),
            pltpu.VMEM((GPW, 16), i32),
            pltpu.VMEM((GPW, 16), f32),
            pltpu.VMEM((GPW, E), i32),
            pltpu.VMEM((NW * GPW, E), i32),
            pltpu.VMEM((GPW, E), i32),
            pltpu.VMEM((E,), i32),
            pltpu.VMEM((E,), i32),
            pltpu.VMEM((NB,), i32),
            pltpu.VMEM((PAD,), i32),
            pltpu.VMEM((PAD,), f32),
            pltpu.VMEM((TPW,), i32),
            pltpu.VMEM((NW, SEG), i32),
            pltpu.VMEM((NW, SEG), f32),
            pltpu.VMEM((SEG,), i32),
            pltpu.VMEM((SEG,), f32),
            pltpu.VMEM((CH,), i32),
            pltpu.VMEM((CH, D), f32),
            pltpu.VMEM_SHARED((NW * GPW, E), i32),
            pltpu.VMEM_SHARED((NW, PAD), i32),
            pltpu.VMEM_SHARED((NW, PAD), f32),
            pltpu.SemaphoreType.DMA,
        ],
    )(_routing_body)
    x_sorted, inv, gate_s, block_expert = sc_route(
        eid2.reshape(N), gate2.reshape(N), h2)

    ys = pl.pallas_call(
        _expert_ffn_body,
        grid_spec=pltpu.PrefetchScalarGridSpec(
            num_scalar_prefetch=1,
            grid=(NB,),
            in_specs=[
                pl.BlockSpec((BT, D), lambda b, be: (b, 0)),
                pl.BlockSpec((1, D, 2 * FF), lambda b, be: (be[b], 0, 0)),
                pl.BlockSpec((1, FF, D), lambda b, be: (be[b], 0, 0)),
                pl.BlockSpec((BT, 1), lambda b, be: (b, 0)),
            ],
            out_specs=pl.BlockSpec((BT, D), lambda b, be: (b, 0)),
        ),
        out_shape=jax.ShapeDtypeStruct((PAD, D), f32),
        interpret=_INTERPRET,
    )(block_expert, x_sorted, Weg, Wed, gate_s.reshape(PAD, 1))

    sc_scatter = functools.partial(
        pl.kernel,
        out_type=jax.ShapeDtypeStruct((N, D), f32),
        mesh=mesh,
        compiler_params=pltpu.CompilerParams(needs_layout_passes=False),
        scratch_types=[
            pltpu.VMEM((N // 32,), i32),
            pltpu.VMEM((CH,), i32),
            pltpu.VMEM((CH, D), f32),
            pltpu.SemaphoreType.DMA,
        ],
    )(_scatter_body)
    y_tok = sc_scatter(ys, inv)

    out = pl.pallas_call(
        _add_body,
        grid=(N // BS,),
        in_specs=[rowblk(D), rowblk(D)],
        out_specs=rowblk(D),
        out_shape=jax.ShapeDtypeStruct((N, D), f32),
        interpret=_INTERPRET,
    )(base, y_tok)
    return out.reshape(1, N, D)


# speedup vs baseline: 1.4860x; 1.0365x over previous
import functools

import jax
import jax.numpy as jnp
from jax.experimental import pallas as pl
from jax.experimental.pallas import tpu as pltpu
from jax.experimental.pallas import tpu_sc as plsc

N = 2048
D = 768
H = 12
DH = 64
E = 64
FF = 512
BT = 64
NB = 96
PAD = NB * BT
BS = 256
BQ = 256

NW = 16
TPW = N // NW
GPW = TPW // 16
SEG = PAD // NW
HALF = PAD // 2
CH = 64

_INTERPRET = False


def _preattn_body(x_ref, w_ref, wq_ref, wk_ref, wv_ref, cos_ref, sin_ref,
                  q_ref, k_ref, v_ref):
    x = x_ref[...]
    var = jnp.mean(x * x, axis=1, keepdims=True)
    h = x * jax.lax.rsqrt(var + 1e-5) * w_ref[...]
    cos = cos_ref[...]
    sin = sin_ref[...]

    def rope(t):
        parts = []
        for hh in range(H):
            th = t[:, hh * DH:(hh + 1) * DH]
            rh = jnp.concatenate([-th[:, DH // 2:], th[:, :DH // 2]], axis=1)
            ch = cos[:, hh * DH:(hh + 1) * DH]
            sh = sin[:, hh * DH:(hh + 1) * DH]
            parts.append(th * ch + rh * sh)
        return jnp.concatenate(parts, axis=1)

    q = jnp.dot(h, wq_ref[...], preferred_element_type=jnp.float32)
    k = jnp.dot(h, wk_ref[...], preferred_element_type=jnp.float32)
    v = jnp.dot(h, wv_ref[...], preferred_element_type=jnp.float32)
    q_ref[...] = rope(q)
    k_ref[...] = rope(k)
    v_ref[...] = v


def _attn_body(q_ref, k_ref, v_ref, o_ref):
    qi = pl.program_id(1)
    q = q_ref[...]
    k = k_ref[...]
    v = v_ref[...]
    rows = qi * BQ + jax.lax.broadcasted_iota(jnp.int32, (BQ, N), 0)
    cols = jax.lax.broadcasted_iota(jnp.int32, (BQ, N), 1)
    mask = cols <= rows
    outs = []
    for hh in range(2):
        qh = q[:, hh * DH:(hh + 1) * DH].astype(jnp.bfloat16)
        kh = k[:, hh * DH:(hh + 1) * DH].astype(jnp.bfloat16)
        vh = v[:, hh * DH:(hh + 1) * DH].astype(jnp.bfloat16)
        s = jax.lax.dot_general(qh, kh, (((1,), (1,)), ((), ())),
                                preferred_element_type=jnp.float32) * 0.125
        s = jnp.where(mask, s, jnp.float32(-1e30))
        m = jnp.max(s, axis=1, keepdims=True)
        p = jnp.exp(s - m)
        l = jnp.sum(p, axis=1, keepdims=True)
        ctx = jax.lax.dot_general(p.astype(jnp.bfloat16), vh,
                                  (((1,), (0,)), ((), ())),
                                  preferred_element_type=jnp.float32)
        outs.append(ctx / l)
    o_ref[...] = jnp.concatenate(outs, axis=1)


def _postattn_body(ctx_ref, x_ref, wo_ref, ln2_ref, wr_ref, wsg_ref, wsd_ref,
                   base_ref, h2_ref, eid_ref, gate_ref):
    xa = x_ref[...] + jnp.dot(ctx_ref[...], wo_ref[...],
                              preferred_element_type=jnp.float32)
    var = jnp.mean(xa * xa, axis=1, keepdims=True)
    h2 = xa * jax.lax.rsqrt(var + 1e-5) * ln2_ref[...]
    logits = jnp.dot(h2, wr_ref[...], preferred_element_type=jnp.float32)
    mx = jnp.max(logits, axis=1, keepdims=True)
    eid_ref[...] = jnp.argmax(logits, axis=1).astype(jnp.int32)[:, None]
    gate_ref[...] = 1.0 / jnp.sum(jnp.exp(logits - mx), axis=1,
                                  keepdims=True)
    gu = jnp.dot(h2, wsg_ref[...], preferred_element_type=jnp.float32)
    g = gu[:, :FF]
    u = gu[:, FF:]
    shared = jnp.dot(jax.nn.silu(g) * u, wsd_ref[...],
                     preferred_element_type=jnp.float32)
    base_ref[...] = xa + shared
    h2_ref[...] = h2


def _expert_ffn_body(be_ref, xs_ref, weg_ref, wed_ref, gate_ref, o_ref):
    del be_ref
    xb = xs_ref[...]
    gu = jnp.dot(xb, weg_ref[0], preferred_element_type=jnp.float32)
    g = gu[:, :FF]
    u = gu[:, FF:]
    y = jnp.dot(jax.nn.silu(g) * u, wed_ref[0],
                preferred_element_type=jnp.float32)
    o_ref[...] = y * gate_ref[...]


def _add_body(a_ref, b_ref, o_ref):
    o_ref[...] = a_ref[...] + b_ref[...]


def _routing_body(eid_hbm, gate_hbm, h2_hbm,
                  xs_hbm, inv_hbm, gout_hbm, be_hbm,
                  eidv, gatev, skv, stv, rkv, sgv, histg, t2v, prv, psv, bsv,
                  bev, tokfull, gatefull, invseg, tmpi, tmpf, desti, destf,
                  cidx, rows, t2s, tokS, gtS, sem):
    i32 = jnp.int32
    c = jax.lax.axis_index("c")
    s = jax.lax.axis_index("s")
    iota = jax.lax.iota(i32, 16)
    zeros16 = jnp.zeros((16,), i32)

    pltpu.sync_copy(eid_hbm.at[pl.ds(s * TPW, TPW)], eidv)
    pltpu.sync_copy(gate_hbm.at[pl.ds(s * TPW, TPW)], gatev)
    for g in range(GPW):
        for k in range(E // 16):
            histg[g, pl.ds(16 * k, 16)] = zeros16
    for g in range(GPW):
        ev = eidv[pl.ds(16 * g, 16)]
        sk, sp = plsc.sort_key_val(ev, iota)
        skv[g, pl.ds(0, 16)] = sk
        tok = s * TPW + 16 * g + sp
        sg = plsc.load_gather(gatev, [sp + 16 * g])
        prev = plsc.load_gather(skv, [jnp.full((16,), g, i32),
                                      jnp.maximum(iota - 1, 0)])
        nxt = plsc.load_gather(skv, [jnp.full((16,), g, i32),
                                     jnp.minimum(iota + 1, 15)])
        bound = (iota == 0) | (sk != prev)
        rst = plsc.cummax(jnp.where(bound, iota, 0))
        rk = iota - rst
        is_last = (iota == 15) | (sk != nxt)
        plsc.store_scatter(histg, [jnp.full((16,), g, i32), sk], rk + 1,
                           mask=is_last)
        stv[g, pl.ds(0, 16)] = tok
        rkv[g, pl.ds(0, 16)] = rk
        sgv[g, pl.ds(0, 16)] = sg
    pltpu.sync_copy(histg, t2s.at[pl.ds(s * GPW, GPW)])

    def zero_body(j, _):
        tokfull[pl.ds(16 * j, 16)] = zeros16
        gatefull[pl.ds(16 * j, 16)] = zeros16.astype(jnp.float32)
        return 0

    jax.lax.fori_loop(0, PAD // 16, zero_body, 0)
    plsc.subcore_barrier()

    pltpu.sync_copy(t2s, t2v)
    zero4 = tuple(jnp.zeros((16,), i32) for _ in range(4))

    def scan_row(r, carry):
        base, tot = carry[:4], carry[4:]
        pred = r < s * GPW
        nb = []
        nt = []
        for k in range(4):
            row = t2v[r, pl.ds(16 * k, 16)]
            nb.append(base[k] + jnp.where(pred, row, 0))
            nt.append(tot[k] + row)
        return tuple(nb) + tuple(nt)

    acc = jax.lax.fori_loop(0, NW * GPW, scan_row, zero4 + zero4)
    base, ctot = list(acc[:4]), list(acc[4:])
    for g in range(GPW):
        for k in range(4):
            prv[g, pl.ds(16 * k, 16)] = base[k]
            base[k] = base[k] + t2v[s * GPW + g, pl.ds(16 * k, 16)]

    carry = jnp.zeros((), i32)
    for k in range(4):
        padk = ((ctot[k] + (BT - 1)) >> 6) << 6
        incl = plsc.cumsum(padk)
        excl = incl - padk + carry
        psv[pl.ds(16 * k, 16)] = excl
        bsv[pl.ds(16 * k, 16)] = excl >> 6
        carry = carry + jnp.sum(padk)

    for g in range(GPW):
        ev = skv[g, pl.ds(0, 16)]
        slot = (plsc.load_gather(psv, [ev])
                + plsc.load_gather(prv, [jnp.full((16,), g, i32), ev])
                + rkv[g, pl.ds(0, 16)])
        tok = stv[g, pl.ds(0, 16)]
        plsc.store_scatter(tokfull, [slot], tok + 1)
        plsc.store_scatter(gatefull, [slot], sgv[g, pl.ds(0, 16)])
        plsc.store_scatter(invseg, [tok - s * TPW], slot)
    pltpu.sync_copy(invseg, inv_hbm.at[pl.ds(s * TPW, TPW)])
    pltpu.sync_copy(tokfull, tokS.at[s])
    pltpu.sync_copy(gatefull, gtS.at[s])

    @pl.when((c == 0) & (s == 0))
    def _():
        def be_row(e, carry):
            bs_e = plsc.load_gather(bsv, [jnp.full((16,), e, i32)])
            return tuple(carry[kb] + (bs_e <= (iota + 16 * kb)).astype(i32)
                         for kb in range(NB // 16))

        cnt = jax.lax.fori_loop(
            0, E, be_row, tuple(jnp.zeros((16,), i32)
                                for _ in range(NB // 16)))
        for kb in range(NB // 16):
            bev[pl.ds(16 * kb, 16)] = jnp.minimum(cnt[kb] - 1, E - 1)
        pltpu.sync_copy(bev, be_hbm)

    plsc.subcore_barrier()

    segsz = HALF // NW
    pltpu.sync_copy(tokS.at[:, pl.ds(SEG * s, SEG)], tmpi)
    pltpu.sync_copy(gtS.at[:, pl.ds(SEG * s, SEG)], tmpf)
    for j in range(SEG // 16):
        acc_i = zeros16
        acc_f = zeros16.astype(jnp.float32)
        for w in range(NW):
            acc_i = acc_i + tmpi[w, pl.ds(16 * j, 16)]
            acc_f = acc_f + tmpf[w, pl.ds(16 * j, 16)]
        slotvec = SEG * s + 16 * j + iota
        desti[pl.ds(16 * j, 16)] = jnp.where(acc_i == 0, slotvec & (N - 1),
                                             acc_i - 1)
        destf[pl.ds(16 * j, 16)] = acc_f
    off = segsz * c
    seg = SEG * s + off
    pltpu.sync_copy(destf.at[pl.ds(off, segsz)],
                    gout_hbm.at[pl.ds(seg, segsz)])
    for ch in range(segsz // CH):
        for k in range(CH // 16):
            cidx[pl.ds(16 * k, 16)] = desti[pl.ds(off + CH * ch + 16 * k, 16)]
        pltpu.async_copy(h2_hbm.at[cidx], rows, sem).wait()
        pltpu.sync_copy(rows, xs_hbm.at[pl.ds(seg + CH * ch, CH)])


def _scatter_body(ys_hbm, inv_hbm, yt_hbm, invv, cidx, rows, sem):
    c = jax.lax.axis_index("c")
    s = jax.lax.axis_index("s")
    base = (16 * c + s) * (N // 32)
    pltpu.sync_copy(inv_hbm.at[pl.ds(base, N // 32)], invv)
    for ch in range((N // 32) // CH):
        for k in range(CH // 16):
            cidx[pl.ds(16 * k, 16)] = invv[pl.ds(CH * ch + 16 * k, 16)]
        pltpu.async_copy(ys_hbm.at[cidx], rows, sem).wait()
        pltpu.sync_copy(rows, yt_hbm.at[pl.ds(base + CH * ch, CH)])


def kernel(x, position_ids, ln1_w, ln2_w, Wq, Wk, Wv, Wo, Wr, Wsg, Wsd,
           Weg, Wed):
    f32 = jnp.float32
    i32 = jnp.int32
    xf = x.reshape(N, D)

    inv_freq = 1.0 / (10000.0 ** (jnp.arange(0, DH, 2, dtype=f32) / DH))
    freqs = position_ids.reshape(N, 1).astype(f32) * inv_freq[None, :]
    emb = jnp.concatenate([freqs, freqs], axis=1)
    cos_t = jnp.tile(jnp.cos(emb), (1, H))
    sin_t = jnp.tile(jnp.sin(emb), (1, H))

    full = lambda shape: pl.BlockSpec(shape, lambda i: (0,) * len(shape))
    rowblk = lambda w: pl.BlockSpec((BS, w), lambda i: (i, 0))

    q, k, v = pl.pallas_call(
        _preattn_body,
        grid=(N // BS,),
        in_specs=[rowblk(D), full((1, D)), full((D, D)), full((D, D)),
                  full((D, D)), rowblk(D), rowblk(D)],
        out_specs=[rowblk(D)] * 3,
        out_shape=[jax.ShapeDtypeStruct((N, D), f32)] * 3,
        interpret=_INTERPRET,
    )(xf, ln1_w.reshape(1, D), Wq, Wk, Wv, cos_t, sin_t)

    ctx = pl.pallas_call(
        _attn_body,
        grid=(H // 2, N // BQ),
        in_specs=[
            pl.BlockSpec((BQ, 2 * DH), lambda p, i: (i, p)),
            pl.BlockSpec((N, 2 * DH), lambda p, i: (0, p)),
            pl.BlockSpec((N, 2 * DH), lambda p, i: (0, p)),
        ],
        out_specs=pl.BlockSpec((BQ, 2 * DH), lambda p, i: (i, p)),
        out_shape=jax.ShapeDtypeStruct((N, D), f32),
        interpret=_INTERPRET,
    )(q, k, v)

    base, h2, eid2, gate2 = pl.pallas_call(
        _postattn_body,
        grid=(N // BS,),
        in_specs=[rowblk(D), rowblk(D), full((D, D)), full((1, D)),
                  full((D, E)), full((D, 2 * FF)), full((FF, D))],
        out_specs=[rowblk(D), rowblk(D), rowblk(1), rowblk(1)],
        out_shape=[jax.ShapeDtypeStruct((N, D), f32),
                   jax.ShapeDtypeStruct((N, D), f32),
                   jax.ShapeDtypeStruct((N, 1), i32),
                   jax.ShapeDtypeStruct((N, 1), f32)],
        interpret=_INTERPRET,
    )(ctx, xf, Wo, ln2_w.reshape(1, D), Wr, Wsg, Wsd)

    mesh = plsc.VectorSubcoreMesh(core_axis_name="c", subcore_axis_name="s")
    sc_route = functools.partial(
        pl.kernel,
        out_type=[jax.ShapeDtypeStruct((PAD, D), f32),
                  jax.ShapeDtypeStruct((N,), i32),
                  jax.ShapeDtypeStruct((PAD,), f32),
                  jax.ShapeDtypeStruct((NB,), i32)],
        mesh=mesh,
        compiler_params=pltpu.CompilerParams(needs_layout_passes=False),
        scratch_types=[
            pltpu.VMEM((TPW,), i32),
            pltpu.VMEM((TPW,), f32),
            pltpu.VMEM((GPW, 16), i32),
            pltpu.VMEM((GPW, 16), i32),
            pltpu.VMEM((GPW, 16), i32),
            pltpu.VMEM((GPW, 16), f32),
            pltpu.VMEM((GPW, E), i32),
            pltpu.VMEM((NW * GPW, E), i32),
            pltpu.VMEM((GPW, E), i32),
            pltpu.VMEM((E,), i32),
            pltpu.VMEM((E,), i32),
            pltpu.VMEM((NB,), i32),
            pltpu.VMEM((PAD,), i32),
            pltpu.VMEM((PAD,), f32),
            pltpu.VMEM((TPW,), i32),
            pltpu.VMEM((NW, SEG), i32),
            pltpu.VMEM((NW, SEG), f32),
            pltpu.VMEM((SEG,), i32),
            pltpu.VMEM((SEG,), f32),
            pltpu.VMEM((CH,), i32),
            pltpu.VMEM((CH, D), f32),
            pltpu.VMEM_SHARED((NW * GPW, E), i32),
            pltpu.VMEM_SHARED((NW, PAD), i32),
            pltpu.VMEM_SHARED((NW, PAD), f32),
            pltpu.SemaphoreType.DMA,
        ],
    )(_routing_body)
    x_sorted, inv, gate_s, block_expert = sc_route(
        eid2.reshape(N), gate2.reshape(N), h2)

    ys = pl.pallas_call(
        _expert_ffn_body,
        grid_spec=pltpu.PrefetchScalarGridSpec(
            num_scalar_prefetch=1,
            grid=(NB,),
            in_specs=[
                pl.BlockSpec((BT, D), lambda b, be: (b, 0)),
                pl.BlockSpec((1, D, 2 * FF), lambda b, be: (be[b], 0, 0)),
                pl.BlockSpec((1, FF, D), lambda b, be: (be[b], 0, 0)),
                pl.BlockSpec((BT, 1), lambda b, be: (b, 0)),
            ],
            out_specs=pl.BlockSpec((BT, D), lambda b, be: (b, 0)),
        ),
        out_shape=jax.ShapeDtypeStruct((PAD, D), f32),
        interpret=_INTERPRET,
    )(block_expert, x_sorted, Weg, Wed, gate_s.reshape(PAD, 1))

    sc_scatter = functools.partial(
        pl.kernel,
        out_type=jax.ShapeDtypeStruct((N, D), f32),
        mesh=mesh,
        compiler_params=pltpu.CompilerParams(needs_layout_passes=False),
        scratch_types=[
            pltpu.VMEM((N // 32,), i32),
            pltpu.VMEM((CH,), i32),
            pltpu.VMEM((CH, D), f32),
            pltpu.SemaphoreType.DMA,
        ],
    )(_scatter_body)
    y_tok = sc_scatter(ys, inv)

    out = pl.pallas_call(
        _add_body,
        grid=(N // BS,),
        in_specs=[rowblk(D), rowblk(D)],
        out_specs=rowblk(D),
        out_shape=jax.ShapeDtypeStruct((N, D), f32),
        interpret=_INTERPRET,
    )(base, y_tok)
    return out.reshape(1, N, D)
